# baseline (speedup 1.0000x reference)
import jax
import jax.numpy as jnp
from jax.experimental import pallas as pl

B = 8
S = 100
N = 64
D = 256
K = 10
IT = 15
M = B * S
CH = 12
MP = 804
G = MP // CH
R = CH * N
KL = 128


def _kmeans_block(x_ref, cid_ref, ctr_ref, loss_ref):
    i = pl.program_id(0)
    X = x_ref[...]
    Xa = jnp.concatenate([X, jnp.ones((R, 1), jnp.float32)], axis=1)
    x_sq = jnp.sum(X * X, axis=1, keepdims=True)

    lane_g = jax.lax.broadcasted_iota(jnp.int32, (R, KL), 1)
    row_r = jax.lax.broadcasted_iota(jnp.int32, (R, KL), 0)
    valid = (lane_g < CH * K) & ((lane_g // K) == (row_r // N))
    neg = jnp.float32(-1e30)

    c0 = jnp.concatenate(
        [X[ci * N:ci * N + K] for ci in range(CH)]
        + [jnp.zeros((KL - CH * K, D), jnp.float32)], axis=0)

    def step(centers):
        c_sq = jnp.sum(centers * centers, axis=1)
        dot = jax.lax.dot_general(X, centers,
                                  (((1,), (1,)), ((), ())))
        sim = 2.0 * dot - x_sq - c_sq[None, :]
        sim = jnp.where(valid, sim, neg)
        cid = jnp.argmax(sim, axis=1)
        onehot = (lane_g == cid[:, None]).astype(jnp.float32)
        sums_aug = jax.lax.dot_general(
            onehot, Xa, (((0,), (0,)), ((), ())))
        counts = sums_aug[:, D:D + 1]
        new_c = sums_aug[:, :D] / jnp.maximum(counts, 1.0)
        new_c = jnp.where(counts > 0.0, new_c, centers)
        return new_c, sim, cid, counts

    centers = jax.lax.fori_loop(
        0, IT, lambda _, c: step(c)[0], c0)

    _, sim, cid, counts = step(centers)

    cid_ref[...] = (cid % K).astype(jnp.int32)[:, None]
    ctr_ref[...] = centers[:CH * K, :]

    row1 = jax.lax.broadcasted_iota(jnp.int32, (R, 1), 0)
    valid_row = (i * CH + row1 // N) < M
    best = jnp.max(sim, axis=1, keepdims=True)
    mds = jnp.maximum(-best, 0.0)
    kml = jnp.sum(jnp.where(valid_row, mds, 0.0)) / float(M * N)

    g1 = jax.lax.broadcasted_iota(jnp.int32, (KL, 1), 0)
    valid_g = (g1 < CH * K) & ((i * CH + g1 // K) < M)
    frac = counts / float(N)
    uni = jnp.sum(jnp.where(valid_g, (frac - 1.0 / K) ** 2, 0.0)) / float(M * K)

    p_r = jax.lax.broadcasted_iota(jnp.int32, (8, 128), 0)
    p_l = jax.lax.broadcasted_iota(jnp.int32, (8, 128), 1)
    acc = (kml * ((p_r == 0) & (p_l == 0)).astype(jnp.float32)
           + uni * ((p_r == 0) & (p_l == 1)).astype(jnp.float32))

    @pl.when(i == 0)
    def _():
        loss_ref[...] = jnp.zeros((8, 128), jnp.float32)

    loss_ref[...] += acc


def kernel(feature):
    x = feature.reshape(M, N, D)
    x = jnp.concatenate([x, x[:MP - M]], axis=0)
    x = x.reshape(MP * N, D)

    cid_flat, ctr_flat, loss = pl.pallas_call(
        _kmeans_block,
        grid=(G,),
        in_specs=[pl.BlockSpec((R, D), lambda i: (i, 0))],
        out_specs=[
            pl.BlockSpec((R, 1), lambda i: (i, 0)),
            pl.BlockSpec((CH * K, D), lambda i: (i, 0)),
            pl.BlockSpec((8, 128), lambda i: (0, 0)),
        ],
        out_shape=[
            jax.ShapeDtypeStruct((MP * N, 1), jnp.int32),
            jax.ShapeDtypeStruct((MP * K, D), jnp.float32),
            jax.ShapeDtypeStruct((8, 128), jnp.float32),
        ],
    )(x)

    cid = cid_flat.reshape(MP, N)[:M].reshape(B, S, N)
    centers = ctr_flat.reshape(MP, K, D)[:M].reshape(B, S, K, D)
    return (cid, centers, loss[0, 0], loss[0, 1])


# speedup vs baseline: 3.5063x; 3.5063x over previous
import jax
import jax.numpy as jnp
from jax.experimental import pallas as pl

B = 8
S = 100
N = 64
D = 256
K = 10
IT = 15
M = B * S
CH = 12
MP = 804
G = MP // CH
R = CH * N
KL = 128


def _kmeans_block(x_ref, cid_ref, ctr_ref, loss_ref):
    i = pl.program_id(0)
    X = x_ref[...]
    Xa = jnp.concatenate([X, jnp.ones((R, 1), jnp.float32)], axis=1)
    x_sq = jnp.sum(X * X, axis=1, keepdims=True)

    lane_g = jax.lax.broadcasted_iota(jnp.int32, (R, KL), 1)
    row_r = jax.lax.broadcasted_iota(jnp.int32, (R, KL), 0)
    valid = (lane_g < CH * K) & ((lane_g // K) == (row_r // N))
    neg = jnp.float32(-1e30)

    c0 = jnp.concatenate(
        [X[ci * N:ci * N + K] for ci in range(CH)]
        + [jnp.zeros((KL - CH * K, D), jnp.float32)], axis=0)

    def step(centers):
        c_sq = jnp.sum(centers * centers, axis=1)
        dot = jax.lax.dot_general(X, centers,
                                  (((1,), (1,)), ((), ())))
        sim = 2.0 * dot - x_sq - c_sq[None, :]
        sim = jnp.where(valid, sim, neg)
        cid = jnp.argmax(sim, axis=1)
        sub_g = jax.lax.broadcasted_iota(jnp.int32, (KL, R), 0)
        onehot_t = (sub_g == cid[None, :]).astype(jnp.float32)
        sums_aug = jax.lax.dot_general(
            onehot_t, Xa, (((1,), (0,)), ((), ())))
        counts = sums_aug[:, D:D + 1]
        new_c = sums_aug[:, :D] / jnp.maximum(counts, 1.0)
        new_c = jnp.where(counts > 0.0, new_c, centers)
        return new_c, sim, cid, counts

    def cond(carry):
        it, _, _, conv = carry
        return (it < IT) & jnp.logical_not(conv)

    def body(carry):
        it, c, prev_cid, _ = carry
        new_c, _, cid, _ = step(c)
        conv = jnp.sum((cid != prev_cid).astype(jnp.float32)) == 0.0
        return it + 1, new_c, cid, conv

    _, centers, _, _ = jax.lax.while_loop(
        cond, body, (jnp.int32(0), c0, jnp.full((R,), -1, jnp.int32),
                     jnp.bool_(False)))

    _, sim, cid, counts = step(centers)

    cid_ref[...] = (cid % K).astype(jnp.int32)[:, None]
    ctr_ref[...] = centers[:CH * K, :]

    row1 = jax.lax.broadcasted_iota(jnp.int32, (R, 1), 0)
    valid_row = (i * CH + row1 // N) < M
    best = jnp.max(sim, axis=1, keepdims=True)
    mds = jnp.maximum(-best, 0.0)
    kml = jnp.sum(jnp.where(valid_row, mds, 0.0)) / float(M * N)

    g1 = jax.lax.broadcasted_iota(jnp.int32, (KL, 1), 0)
    valid_g = (g1 < CH * K) & ((i * CH + g1 // K) < M)
    frac = counts / float(N)
    uni = jnp.sum(jnp.where(valid_g, (frac - 1.0 / K) ** 2, 0.0)) / float(M * K)

    p_r = jax.lax.broadcasted_iota(jnp.int32, (8, 128), 0)
    p_l = jax.lax.broadcasted_iota(jnp.int32, (8, 128), 1)
    acc = (kml * ((p_r == 0) & (p_l == 0)).astype(jnp.float32)
           + uni * ((p_r == 0) & (p_l == 1)).astype(jnp.float32))

    @pl.when(i == 0)
    def _():
        loss_ref[...] = jnp.zeros((8, 128), jnp.float32)

    loss_ref[...] += acc


def kernel(feature):
    x = feature.reshape(M, N, D)
    x = jnp.concatenate([x, x[:MP - M]], axis=0)
    x = x.reshape(MP * N, D)

    cid_flat, ctr_flat, loss = pl.pallas_call(
        _kmeans_block,
        grid=(G,),
        in_specs=[pl.BlockSpec((R, D), lambda i: (i, 0))],
        out_specs=[
            pl.BlockSpec((R, 1), lambda i: (i, 0)),
            pl.BlockSpec((CH * K, D), lambda i: (i, 0)),
            pl.BlockSpec((8, 128), lambda i: (0, 0)),
        ],
        out_shape=[
            jax.ShapeDtypeStruct((MP * N, 1), jnp.int32),
            jax.ShapeDtypeStruct((MP * K, D), jnp.float32),
            jax.ShapeDtypeStruct((8, 128), jnp.float32),
        ],
    )(x)

    cid = cid_flat.reshape(MP, N)[:M].reshape(B, S, N)
    centers = ctr_flat.reshape(MP, K, D)[:M].reshape(B, S, K, D)
    return (cid, centers, loss[0, 0], loss[0, 1])
